# Initial kernel scaffold; baseline (speedup 1.0000x reference)
import jax
import jax.numpy as jnp
from jax.experimental import pallas as pl


def kernel(x, W_gate):
    raise NotImplementedError("write your pallas kernel here")


# speedup vs baseline: 1.2416x; 1.2416x over previous
import jax
import jax.numpy as jnp
from jax.experimental import pallas as pl

N_TOKENS = 8192
D_MODEL = 4096
BLK = 256


def _router_kernel(x_ref, wg_ref, x0_ref, x1_ref, xout_ref):
    x = x_ref[...]
    logits = jnp.dot(x, wg_ref[...])
    score = jax.nn.softmax(logits, axis=-1)
    s0 = score[:, 0:1]
    s1 = score[:, 1:2]
    take0 = s0 >= s1
    w0 = jnp.where(take0, s0, 0.0)
    w1 = jnp.where(take0, 0.0, s1)
    x0_ref[...] = x * w0
    x1_ref[...] = x * w1
    xout_ref[...] = x * (w0 + w1)


@jax.jit
def kernel(x, W_gate):
    grid = (N_TOKENS // BLK,)
    out = pl.pallas_call(
        _router_kernel,
        grid=grid,
        in_specs=[
            pl.BlockSpec((BLK, D_MODEL), lambda i: (i, 0)),
            pl.BlockSpec((D_MODEL, 2), lambda i: (0, 0)),
        ],
        out_specs=[
            pl.BlockSpec((BLK, D_MODEL), lambda i: (i, 0)),
            pl.BlockSpec((BLK, D_MODEL), lambda i: (i, 0)),
            pl.BlockSpec((BLK, D_MODEL), lambda i: (i, 0)),
        ],
        out_shape=[
            jax.ShapeDtypeStruct((N_TOKENS, D_MODEL), x.dtype),
            jax.ShapeDtypeStruct((N_TOKENS, D_MODEL), x.dtype),
            jax.ShapeDtypeStruct((N_TOKENS, D_MODEL), x.dtype),
        ],
    )(x, W_gate)
    return (out[0], out[1], out[2])
